# Initial kernel scaffold; baseline (speedup 1.0000x reference)
import jax
import jax.numpy as jnp
from jax.experimental import pallas as pl


def kernel(C, F, A, SP1, SP0, params, edge_index):
    raise NotImplementedError("write your pallas kernel here")


# speedup vs baseline: 2.2539x; 2.2539x over previous
import functools

import jax
import jax.numpy as jnp
from jax import lax
from jax.experimental import pallas as pl
from jax.experimental.pallas import tpu as pltpu
from jax.experimental.pallas import tpu_sc as plsc

F32 = jnp.float32

_BE = 6400
_BN = 5000
_CH = 128
_NC = 2
_NS = 16
_NW = _NC * _NS


def _mm(x, w):
    return jnp.dot(x, w, preferred_element_type=F32)


def _relu(x):
    return jnp.maximum(x, 0.0)


def _sig(x):
    return jax.nn.sigmoid(x)


def _tc_call(body, n_rows, block_rows, data_ins, weight_ins, out_widths):
    grid = n_rows // block_rows
    in_specs = (
        [pl.BlockSpec((block_rows, a.shape[1]), lambda i: (i, 0))
         for a in data_ins]
        + [pl.BlockSpec(w.shape, lambda i: (0, 0)) for w in weight_ins]
    )
    out_specs = [pl.BlockSpec((block_rows, w), lambda i: (i, 0))
                 for w in out_widths]
    out_shape = [jax.ShapeDtypeStruct((n_rows, w), F32) for w in out_widths]
    return pl.pallas_call(
        body, grid=(grid,), in_specs=in_specs, out_specs=out_specs,
        out_shape=out_shape,
    )(*data_ins, *weight_ins)


def _edge1_body(a, sp1, sp0, gs, gd,
                ew1, eb1, ew2, eb2, ew3, eb3,
                we, b1, w2, b2, w3, b3, wh,
                e1_o, zlo_o, zhi_o):
    x = _relu(a[...] * ew1[0:1, :] + sp1[...] * ew1[1:2, :]
              + sp0[...] * ew1[2:3, :] + eb1[...])
    x = _relu(_mm(x, ew2[...]) + eb2[...])
    e0 = _sig(_mm(x, ew3[...]) + eb3[...])
    h = _relu(_mm(e0, we[...]) + gs[...] + gd[...] + b1[...])
    h = _relu(_mm(h, w2[...]) + b2[...])
    e1 = _sig(_mm(h, w3[...]) + b3[...])
    e1_o[...] = e1
    z = _mm(e1, wh[...])
    zlo_o[...] = z[:, :8]
    zhi_o[...] = z[:, 8:]


def _edge2_body(e_in, gs, gd, we, b1, w2, b2, w3, b3, wh,
                e_o, zlo_o, zhi_o):
    h = _relu(_mm(e_in[...], we[...]) + gs[...] + gd[...] + b1[...])
    h = _relu(_mm(h, w2[...]) + b2[...])
    e2 = _sig(_mm(h, w3[...]) + b3[...])
    e_o[...] = e2
    z = _mm(e2, wh[...])
    zlo_o[...] = z[:, :8]
    zhi_o[...] = z[:, 8:]


def _dec_body(e_in, gqs, gqd, wde, bd1, wd2, bd2, wd3, bd3, p_o):
    d = _relu(_mm(e_in[...], wde[...]) + gqs[...] + gqd[...] + bd1[...])
    d = _relu(_mm(d, wd2[...]) + bd2[...])
    p_o[...] = _sig(_mm(d, wd3[...]) + bd3[...])


def _nodeA_body(c, f, nw1, nb1, nw2, nb2, nw3, nb3, ws, wd,
                n0_o, ps_o, pd_o):
    x = _relu(c[...] * nw1[0:1, :] + f[...] * nw1[1:2, :] + nb1[...])
    x = _relu(_mm(x, nw2[...]) + nb2[...])
    n0 = _sig(_mm(x, nw3[...]) + nb3[...])
    n0_o[...] = n0
    ps_o[...] = _mm(n0, ws[...])
    pd_o[...] = _mm(n0, wd[...])


def _nodeU_body(n_in, sa, sb, da, db, wn, b1, w2, b2, w3, b3, ws, wd,
                n_o, ps_o, pd_o):
    dg = jnp.maximum(da[...] + db[...], 1.0)
    hw = jnp.concatenate([sa[...] / dg, sb[...] / dg], axis=1)
    g = _relu(_mm(n_in[...], wn[...]) + hw + b1[...])
    g = _relu(_mm(g, w2[...]) + b2[...])
    nn = _sig(_mm(g, w3[...]) + b3[...])
    n_o[...] = nn
    ps_o[...] = _mm(nn, ws[...])
    pd_o[...] = _mm(nn, wd[...])


def _sc_mesh():
    return plsc.VectorSubcoreMesh(core_axis_name="c", subcore_axis_name="s",
                                  num_cores=_NC, num_subcores=_NS)


_SC_PARAMS = pltpu.CompilerParams(use_tc_tiling_on_sc=False)


@functools.lru_cache(maxsize=None)
def _make_gather(n_edges, width, k):
    nchunks = n_edges // _CH
    nblocks = nchunks // k
    max_iters = (nblocks + _NW - 1) // _NW

    @functools.partial(
        pl.kernel, mesh=_sc_mesh(), compiler_params=_SC_PARAMS,
        out_type=(jax.ShapeDtypeStruct((nchunks, _CH, width), F32),
                  jax.ShapeDtypeStruct((nchunks, _CH, width), F32)),
        scratch_types=[
            pltpu.VMEM((k, _CH), jnp.int32),
            pltpu.VMEM((k, _CH), jnp.int32),
            pltpu.VMEM((k, _CH, width), F32),
            pltpu.VMEM((k, _CH, width), F32),
            pltpu.SemaphoreType.DMA,
        ],
    )
    def gather_k(ts_hbm, td_hbm, src_hbm, dst_hbm, gs_hbm, gd_hbm,
                 i1, i2, g1, g2, sem):
        wid = lax.axis_index("s") * _NC + lax.axis_index("c")

        @pl.loop(0, max_iters)
        def _(it):
            b = wid + it * _NW

            @pl.when(b < nblocks)
            def _():
                row = b * k
                pltpu.sync_copy(src_hbm.at[pl.ds(row, k), :], i1)
                pltpu.sync_copy(dst_hbm.at[pl.ds(row, k), :], i2)
                cps = []
                for j in range(k):
                    cps.append(
                        pltpu.async_copy(ts_hbm.at[i1.at[j]], g1.at[j], sem))
                    cps.append(
                        pltpu.async_copy(td_hbm.at[i2.at[j]], g2.at[j], sem))
                for cp in cps:
                    cp.wait()
                pltpu.sync_copy(g1, gs_hbm.at[pl.ds(row, k)])
                pltpu.sync_copy(g2, gd_hbm.at[pl.ds(row, k)])

    return gather_k


@functools.lru_cache(maxsize=None)
def _make_scatter(n_nodes, n_edges, k):
    nchunks = n_edges // _CH
    nblocks = nchunks // k
    max_iters = (nblocks + _NS - 1) // _NS
    rpt = n_nodes // _NS

    @functools.partial(
        pl.kernel, mesh=_sc_mesh(), compiler_params=_SC_PARAMS,
        out_type=jax.ShapeDtypeStruct((_NC, n_nodes, 8), F32),
        scratch_types=[
            pltpu.VMEM((k, _CH), jnp.int32),
            pltpu.VMEM((k, _CH, 8), F32),
            pltpu.VMEM_SHARED((n_nodes, 8), F32),
            pltpu.SemaphoreType.DMA,
        ],
    )
    def scatter_k(zlo_hbm, zhi_hbm, dst_hbm, zero_hbm, out_hbm,
                  idx, pay, acc, sem):
        cid = lax.axis_index("c")
        tid = lax.axis_index("s")
        pltpu.sync_copy(zero_hbm.at[pl.ds(tid * rpt, rpt), :],
                        acc.at[pl.ds(tid * rpt, rpt), :])
        plsc.subcore_barrier()

        @pl.loop(0, max_iters)
        def _(it):
            b = tid + it * _NS

            @pl.when(b < nblocks)
            def _():
                row = b * k
                pltpu.sync_copy(dst_hbm.at[pl.ds(row, k), :], idx)

                @pl.when(cid == 0)
                def _():
                    pltpu.sync_copy(zlo_hbm.at[pl.ds(row, k)], pay)

                @pl.when(cid == 1)
                def _():
                    pltpu.sync_copy(zhi_hbm.at[pl.ds(row, k)], pay)

                cps = [pltpu.async_copy(pay.at[j], acc.at[idx.at[j]], sem,
                                        add=True)
                       for j in range(k)]
                for cp in cps:
                    cp.wait()

        plsc.subcore_barrier()
        pltpu.sync_copy(acc.at[pl.ds(tid * rpt, rpt), :],
                        out_hbm.at[cid, pl.ds(tid * rpt, rpt), :])

    return scatter_k


@functools.lru_cache(maxsize=None)
def _make_degree(n_nodes, n_edges, k):
    nchunks = n_edges // _CH
    nblocks = nchunks // k
    hblocks = nblocks // _NC
    max_iters = (hblocks + _NS - 1) // _NS
    rpt = n_nodes // _NS

    @functools.partial(
        pl.kernel, mesh=_sc_mesh(), compiler_params=_SC_PARAMS,
        out_type=jax.ShapeDtypeStruct((_NC, n_nodes, 8), F32),
        scratch_types=[
            pltpu.VMEM((k, _CH), jnp.int32),
            pltpu.VMEM((_CH, 8), F32),
            pltpu.VMEM_SHARED((n_nodes, 8), F32),
            pltpu.SemaphoreType.DMA,
        ],
    )
    def degree_k(dst_hbm, zero_hbm, ones_hbm, out_hbm, idx, ones, acc, sem):
        cid = lax.axis_index("c")
        tid = lax.axis_index("s")
        pltpu.sync_copy(zero_hbm.at[pl.ds(tid * rpt, rpt), :],
                        acc.at[pl.ds(tid * rpt, rpt), :])
        pltpu.sync_copy(ones_hbm, ones)
        plsc.subcore_barrier()

        @pl.loop(0, max_iters)
        def _(it):
            b = cid * hblocks + tid + it * _NS

            @pl.when(b < (cid + 1) * hblocks)
            def _():
                row = b * k
                pltpu.sync_copy(dst_hbm.at[pl.ds(row, k), :], idx)
                cps = [pltpu.async_copy(ones, acc.at[idx.at[j]], sem,
                                        add=True)
                       for j in range(k)]
                for cp in cps:
                    cp.wait()

        plsc.subcore_barrier()
        pltpu.sync_copy(acc.at[pl.ds(tid * rpt, rpt), :],
                        out_hbm.at[cid, pl.ds(tid * rpt, rpt), :])

    return degree_k


def _b(bias):
    return bias.reshape(1, -1)


def kernel(C, F, A, SP1, SP0, params, edge_index):
    n_nodes = C.shape[0]
    n_edges = A.shape[0]
    nchunks = n_edges // _CH
    src = edge_index[0].astype(jnp.int32)
    dst = edge_index[1].astype(jnp.int32)
    src2 = src.reshape(nchunks, _CH)
    dst2 = dst.reshape(nchunks, _CH)
    zero8 = jnp.zeros((n_nodes, 8), F32)
    ones8 = jnp.ones((_CH, 8), F32)

    (enW1, enb1), (enW2, enb2), (enW3, enb3) = params['enc_n']
    (eeW1, eeb1), (eeW2, eeb2), (eeW3, eeb3) = params['enc_e']
    (c1eW1, c1eb1), (c1eW2, c1eb2), (c1eW3, c1eb3) = params['c1_e']
    (c1nW1, c1nb1), (c1nW2, c1nb2), (c1nW3, c1nb3) = params['c1_n']
    (c3eW1, c3eb1), (c3eW2, c3eb2), (c3eW3, c3eb3) = params['c3_e']
    (c3nW1, c3nb1), (c3nW2, c3nb2), (c3nW3, c3nb3) = params['c3_n']
    (dW1, db1), (dW2, db2), (dW3, db3) = params['dec']

    c1e_We, c1e_Ws, c1e_Wd = c1eW1[:64], c1eW1[64:128], c1eW1[128:]
    c3e_We, c3e_Ws, c3e_Wd = c3eW1[:64], c3eW1[64:128], c3eW1[128:]
    c1n_Wn, c1n_Wh = c1nW1[:64], c1nW1[64:]
    c3n_Wn, c3n_Wh = c3nW1[:64], c3nW1[64:]
    d_We, d_Ws, d_Wd = dW1[:64], dW1[64:128], dW1[128:]

    gather16 = _make_gather(n_edges, 16, 10)
    gather64 = _make_gather(n_edges, 64, 4)
    scatter = _make_scatter(n_nodes, n_edges, 10)
    degree = _make_degree(n_nodes, n_edges, 10)

    deg = degree(dst2, zero8, ones8)
    dega, degb = deg[0], deg[1]

    n0, ps1, pd1 = _tc_call(
        _nodeA_body, n_nodes, _BN, [C, F],
        [enW1, _b(enb1), enW2, _b(enb2), enW3, _b(enb3), c1e_Ws, c1e_Wd],
        [64, 16, 16])

    gs1, gd1 = gather16(ps1, pd1, src2, dst2)

    e1, z1lo, z1hi = _tc_call(
        _edge1_body, n_edges, _BE,
        [A, SP1, SP0,
         gs1.reshape(n_edges, 16), gd1.reshape(n_edges, 16)],
        [eeW1, _b(eeb1), eeW2, _b(eeb2), eeW3, _b(eeb3),
         c1e_We, _b(c1eb1), c1eW2, _b(c1eb2), c1eW3, _b(c1eb3), c1n_Wh],
        [64, 8, 8])

    s1 = scatter(z1lo.reshape(nchunks, _CH, 8),
                 z1hi.reshape(nchunks, _CH, 8), dst2, zero8)

    n1, ps2, pd2 = _tc_call(
        _nodeU_body, n_nodes, _BN, [n0, s1[0], s1[1], dega, degb],
        [c1n_Wn, _b(c1nb1), c1nW2, _b(c1nb2), c1nW3, _b(c1nb3),
         c3e_Ws, c3e_Wd],
        [64, 16, 16])

    gs2, gd2 = gather16(ps2, pd2, src2, dst2)

    e2, z2lo, z2hi = _tc_call(
        _edge2_body, n_edges, _BE,
        [e1, gs2.reshape(n_edges, 16), gd2.reshape(n_edges, 16)],
        [c3e_We, _b(c3eb1), c3eW2, _b(c3eb2), c3eW3, _b(c3eb3), c3n_Wh],
        [64, 8, 8])

    s2 = scatter(z2lo.reshape(nchunks, _CH, 8),
                 z2hi.reshape(nchunks, _CH, 8), dst2, zero8)

    _n2, qs, qd = _tc_call(
        _nodeU_body, n_nodes, _BN, [n1, s2[0], s2[1], dega, degb],
        [c3n_Wn, _b(c3nb1), c3nW2, _b(c3nb2), c3nW3, _b(c3nb3),
         d_Ws, d_Wd],
        [64, 64, 64])

    gqs, gqd = gather64(qs, qd, src2, dst2)

    (p_out,) = _tc_call(
        _dec_body, n_edges, _BE,
        [e2, gqs.reshape(n_edges, 64), gqd.reshape(n_edges, 64)],
        [d_We, _b(db1), dW2, _b(db2), dW3, _b(db3)],
        [1])

    return p_out.reshape(n_edges)


# speedup vs baseline: 2.4943x; 1.1067x over previous
import functools

import jax
import jax.numpy as jnp
from jax import lax
from jax.experimental import pallas as pl
from jax.experimental.pallas import tpu as pltpu
from jax.experimental.pallas import tpu_sc as plsc

F32 = jnp.float32

_BE = 6400
_BN = 5000
_CH = 128
_NC = 2
_NS = 16
_NW = _NC * _NS


def _mm(x, w):
    return jnp.dot(x, w, preferred_element_type=F32)


def _relu(x):
    return jnp.maximum(x, 0.0)


def _sig(x):
    return 0.5 * jnp.tanh(0.5 * x) + 0.5


def _tc_call(body, n_rows, block_rows, data_ins, weight_ins, out_widths):
    grid = n_rows // block_rows
    in_specs = (
        [pl.BlockSpec((block_rows, a.shape[1]), lambda i: (i, 0))
         for a in data_ins]
        + [pl.BlockSpec(w.shape, lambda i: (0, 0)) for w in weight_ins]
    )
    out_specs = [pl.BlockSpec((block_rows, w), lambda i: (i, 0))
                 for w in out_widths]
    out_shape = [jax.ShapeDtypeStruct((n_rows, w), F32) for w in out_widths]
    return pl.pallas_call(
        body, grid=(grid,), in_specs=in_specs, out_specs=out_specs,
        out_shape=out_shape,
    )(*data_ins, *weight_ins)


def _edge1_body(ax, gs, gd,
                ew1, eb1, ew2, eb2, ew3, eb3,
                we, b1, w2, b2, w3, b3, wh,
                e1_o, zlo_o, zhi_o):
    x = _relu(_mm(ax[...], ew1[...]) + eb1[...])
    x = _relu(_mm(x, ew2[...]) + eb2[...])
    e0 = _sig(_mm(x, ew3[...]) + eb3[...])
    h = _relu(_mm(e0, we[...]) + gs[...] + gd[...] + b1[...])
    h = _relu(_mm(h, w2[...]) + b2[...])
    e1 = _sig(_mm(h, w3[...]) + b3[...])
    e1_o[...] = e1
    z = _mm(e1, wh[...])
    zlo_o[...] = z[:, :8]
    zhi_o[...] = z[:, 8:]


def _edge2_body(e_in, gs, gd, we, b1, w2, b2, w3, b3, wh,
                e_o, zlo_o, zhi_o):
    h = _relu(_mm(e_in[...], we[...]) + gs[...] + gd[...] + b1[...])
    h = _relu(_mm(h, w2[...]) + b2[...])
    e2 = _sig(_mm(h, w3[...]) + b3[...])
    e_o[...] = e2
    z = _mm(e2, wh[...])
    zlo_o[...] = z[:, :8]
    zhi_o[...] = z[:, 8:]


def _dec_body(e_in, gqs, gqd, wde, bd1, wd2, bd2, wd3, bd3, p_o):
    d = _relu(_mm(e_in[...], wde[...]) + gqs[...] + gqd[...] + bd1[...])
    d = _relu(_mm(d, wd2[...]) + bd2[...])
    p_o[...] = _sig(_mm(d, wd3[...]) + bd3[...])


def _nodeA_body(cf, nw1, nb1, nw2, nb2, nw3, nb3, ws, wd,
                n0_o, ps_o, pd_o):
    x = _relu(_mm(cf[...], nw1[...]) + nb1[...])
    x = _relu(_mm(x, nw2[...]) + nb2[...])
    n0 = _sig(_mm(x, nw3[...]) + nb3[...])
    n0_o[...] = n0
    ps_o[...] = _mm(n0, ws[...])
    pd_o[...] = _mm(n0, wd[...])


def _nodeU_body(n_in, sa, sb, da, db, wn, b1, w2, b2, w3, b3, ws, wd,
                n_o, ps_o, pd_o):
    dg = jnp.maximum(da[...] + db[...], 1.0)
    hw = jnp.concatenate([sa[...] / dg, sb[...] / dg], axis=1)
    g = _relu(_mm(n_in[...], wn[...]) + hw + b1[...])
    g = _relu(_mm(g, w2[...]) + b2[...])
    nn = _sig(_mm(g, w3[...]) + b3[...])
    n_o[...] = nn
    ps_o[...] = _mm(nn, ws[...])
    pd_o[...] = _mm(nn, wd[...])


def _sc_mesh():
    return plsc.VectorSubcoreMesh(core_axis_name="c", subcore_axis_name="s",
                                  num_cores=_NC, num_subcores=_NS)


_SC_PARAMS = pltpu.CompilerParams(use_tc_tiling_on_sc=False)


@functools.lru_cache(maxsize=None)
def _make_gather(n_edges, width, k):
    nchunks = n_edges // _CH
    nblocks = nchunks // k
    max_iters = (nblocks + _NW - 1) // _NW

    @functools.partial(
        pl.kernel, mesh=_sc_mesh(), compiler_params=_SC_PARAMS,
        out_type=(jax.ShapeDtypeStruct((nchunks, _CH, width), F32),
                  jax.ShapeDtypeStruct((nchunks, _CH, width), F32)),
        scratch_types=[
            pltpu.VMEM((k, _CH), jnp.int32),
            pltpu.VMEM((k, _CH), jnp.int32),
            pltpu.VMEM((k, _CH, width), F32),
            pltpu.VMEM((k, _CH, width), F32),
            pltpu.SemaphoreType.DMA,
        ],
    )
    def gather_k(ts_hbm, td_hbm, src_hbm, dst_hbm, gs_hbm, gd_hbm,
                 i1, i2, g1, g2, sem):
        wid = lax.axis_index("s") * _NC + lax.axis_index("c")

        @pl.loop(0, max_iters)
        def _(it):
            b = wid + it * _NW

            @pl.when(b < nblocks)
            def _():
                row = b * k
                pltpu.sync_copy(src_hbm.at[pl.ds(row, k), :], i1)
                pltpu.sync_copy(dst_hbm.at[pl.ds(row, k), :], i2)
                cps = []
                for j in range(k):
                    cps.append(
                        pltpu.async_copy(ts_hbm.at[i1.at[j]], g1.at[j], sem))
                    cps.append(
                        pltpu.async_copy(td_hbm.at[i2.at[j]], g2.at[j], sem))
                for cp in cps:
                    cp.wait()
                pltpu.sync_copy(g1, gs_hbm.at[pl.ds(row, k)])
                pltpu.sync_copy(g2, gd_hbm.at[pl.ds(row, k)])

    return gather_k


@functools.lru_cache(maxsize=None)
def _make_scatter(n_nodes, n_edges, k):
    nchunks = n_edges // _CH
    nblocks = nchunks // k
    max_iters = (nblocks + _NS - 1) // _NS
    rpt = n_nodes // _NS

    @functools.partial(
        pl.kernel, mesh=_sc_mesh(), compiler_params=_SC_PARAMS,
        out_type=jax.ShapeDtypeStruct((_NC, n_nodes, 8), F32),
        scratch_types=[
            pltpu.VMEM((k, _CH), jnp.int32),
            pltpu.VMEM((k, _CH, 8), F32),
            pltpu.VMEM_SHARED((n_nodes, 8), F32),
            pltpu.SemaphoreType.DMA,
        ],
    )
    def scatter_k(zlo_hbm, zhi_hbm, dst_hbm, zero_hbm, out_hbm,
                  idx, pay, acc, sem):
        cid = lax.axis_index("c")
        tid = lax.axis_index("s")
        pltpu.sync_copy(zero_hbm.at[pl.ds(tid * rpt, rpt), :],
                        acc.at[pl.ds(tid * rpt, rpt), :])
        plsc.subcore_barrier()

        @pl.loop(0, max_iters)
        def _(it):
            b = tid + it * _NS

            @pl.when(b < nblocks)
            def _():
                row = b * k
                pltpu.sync_copy(dst_hbm.at[pl.ds(row, k), :], idx)

                @pl.when(cid == 0)
                def _():
                    pltpu.sync_copy(zlo_hbm.at[pl.ds(row, k)], pay)

                @pl.when(cid == 1)
                def _():
                    pltpu.sync_copy(zhi_hbm.at[pl.ds(row, k)], pay)

                cps = [pltpu.async_copy(pay.at[j], acc.at[idx.at[j]], sem,
                                        add=True)
                       for j in range(k)]
                for cp in cps:
                    cp.wait()

        plsc.subcore_barrier()
        pltpu.sync_copy(acc.at[pl.ds(tid * rpt, rpt), :],
                        out_hbm.at[cid, pl.ds(tid * rpt, rpt), :])

    return scatter_k


@functools.lru_cache(maxsize=None)
def _make_degree(n_nodes, n_edges, k):
    nchunks = n_edges // _CH
    nblocks = nchunks // k
    hblocks = nblocks // _NC
    max_iters = (hblocks + _NS - 1) // _NS
    rpt = n_nodes // _NS

    @functools.partial(
        pl.kernel, mesh=_sc_mesh(), compiler_params=_SC_PARAMS,
        out_type=jax.ShapeDtypeStruct((_NC, n_nodes, 8), F32),
        scratch_types=[
            pltpu.VMEM((k, _CH), jnp.int32),
            pltpu.VMEM((_CH, 8), F32),
            pltpu.VMEM_SHARED((n_nodes, 8), F32),
            pltpu.SemaphoreType.DMA,
        ],
    )
    def degree_k(dst_hbm, zero_hbm, ones_hbm, out_hbm, idx, ones, acc, sem):
        cid = lax.axis_index("c")
        tid = lax.axis_index("s")
        pltpu.sync_copy(zero_hbm.at[pl.ds(tid * rpt, rpt), :],
                        acc.at[pl.ds(tid * rpt, rpt), :])
        pltpu.sync_copy(ones_hbm, ones)
        plsc.subcore_barrier()

        @pl.loop(0, max_iters)
        def _(it):
            b = cid * hblocks + tid + it * _NS

            @pl.when(b < (cid + 1) * hblocks)
            def _():
                row = b * k
                pltpu.sync_copy(dst_hbm.at[pl.ds(row, k), :], idx)
                cps = [pltpu.async_copy(ones, acc.at[idx.at[j]], sem,
                                        add=True)
                       for j in range(k)]
                for cp in cps:
                    cp.wait()

        plsc.subcore_barrier()
        pltpu.sync_copy(acc.at[pl.ds(tid * rpt, rpt), :],
                        out_hbm.at[cid, pl.ds(tid * rpt, rpt), :])

    return degree_k


def _b(bias):
    return bias.reshape(1, -1)


def kernel(C, F, A, SP1, SP0, params, edge_index):
    n_nodes = C.shape[0]
    n_edges = A.shape[0]
    nchunks = n_edges // _CH
    src = edge_index[0].astype(jnp.int32)
    dst = edge_index[1].astype(jnp.int32)
    src2 = src.reshape(nchunks, _CH)
    dst2 = dst.reshape(nchunks, _CH)
    zero8 = jnp.zeros((n_nodes, 8), F32)
    ones8 = jnp.ones((_CH, 8), F32)

    (enW1, enb1), (enW2, enb2), (enW3, enb3) = params['enc_n']
    (eeW1, eeb1), (eeW2, eeb2), (eeW3, eeb3) = params['enc_e']
    (c1eW1, c1eb1), (c1eW2, c1eb2), (c1eW3, c1eb3) = params['c1_e']
    (c1nW1, c1nb1), (c1nW2, c1nb2), (c1nW3, c1nb3) = params['c1_n']
    (c3eW1, c3eb1), (c3eW2, c3eb2), (c3eW3, c3eb3) = params['c3_e']
    (c3nW1, c3nb1), (c3nW2, c3nb2), (c3nW3, c3nb3) = params['c3_n']
    (dW1, db1), (dW2, db2), (dW3, db3) = params['dec']

    c1e_We, c1e_Ws, c1e_Wd = c1eW1[:64], c1eW1[64:128], c1eW1[128:]
    c3e_We, c3e_Ws, c3e_Wd = c3eW1[:64], c3eW1[64:128], c3eW1[128:]
    c1n_Wn, c1n_Wh = c1nW1[:64], c1nW1[64:]
    c3n_Wn, c3n_Wh = c3nW1[:64], c3nW1[64:]
    d_We, d_Ws, d_Wd = dW1[:64], dW1[64:128], dW1[128:]

    gather16 = _make_gather(n_edges, 16, 10)
    gather64 = _make_gather(n_edges, 64, 4)
    scatter = _make_scatter(n_nodes, n_edges, 10)
    degree = _make_degree(n_nodes, n_edges, 10)

    deg = degree(dst2, zero8, ones8)
    dega, degb = deg[0], deg[1]

    n0, ps1, pd1 = _tc_call(
        _nodeA_body, n_nodes, _BN, [jnp.concatenate([C, F], axis=1)],
        [enW1, _b(enb1), enW2, _b(enb2), enW3, _b(enb3), c1e_Ws, c1e_Wd],
        [64, 16, 16])

    gs1, gd1 = gather16(ps1, pd1, src2, dst2)

    e1, z1lo, z1hi = _tc_call(
        _edge1_body, n_edges, _BE,
        [jnp.concatenate([A, SP1, SP0], axis=1),
         gs1.reshape(n_edges, 16), gd1.reshape(n_edges, 16)],
        [eeW1, _b(eeb1), eeW2, _b(eeb2), eeW3, _b(eeb3),
         c1e_We, _b(c1eb1), c1eW2, _b(c1eb2), c1eW3, _b(c1eb3), c1n_Wh],
        [64, 8, 8])

    s1 = scatter(z1lo.reshape(nchunks, _CH, 8),
                 z1hi.reshape(nchunks, _CH, 8), dst2, zero8)

    n1, ps2, pd2 = _tc_call(
        _nodeU_body, n_nodes, _BN, [n0, s1[0], s1[1], dega, degb],
        [c1n_Wn, _b(c1nb1), c1nW2, _b(c1nb2), c1nW3, _b(c1nb3),
         c3e_Ws, c3e_Wd],
        [64, 16, 16])

    gs2, gd2 = gather16(ps2, pd2, src2, dst2)

    e2, z2lo, z2hi = _tc_call(
        _edge2_body, n_edges, _BE,
        [e1, gs2.reshape(n_edges, 16), gd2.reshape(n_edges, 16)],
        [c3e_We, _b(c3eb1), c3eW2, _b(c3eb2), c3eW3, _b(c3eb3), c3n_Wh],
        [64, 8, 8])

    s2 = scatter(z2lo.reshape(nchunks, _CH, 8),
                 z2hi.reshape(nchunks, _CH, 8), dst2, zero8)

    _n2, qs, qd = _tc_call(
        _nodeU_body, n_nodes, _BN, [n1, s2[0], s2[1], dega, degb],
        [c3n_Wn, _b(c3nb1), c3nW2, _b(c3nb2), c3nW3, _b(c3nb3),
         d_Ws, d_Wd],
        [64, 64, 64])

    gqs, gqd = gather64(qs, qd, src2, dst2)

    (p_out,) = _tc_call(
        _dec_body, n_edges, _BE,
        [e2, gqs.reshape(n_edges, 64), gqd.reshape(n_edges, 64)],
        [d_We, _b(db1), dW2, _b(db2), dW3, _b(db3)],
        [1])

    return p_out.reshape(n_edges)


# speedup vs baseline: 3.1205x; 1.2510x over previous
import functools

import jax
import jax.numpy as jnp
from jax import lax
from jax.experimental import pallas as pl
from jax.experimental.pallas import tpu as pltpu
from jax.experimental.pallas import tpu_sc as plsc

F32 = jnp.float32

_BE = 6400
_BN = 5000
_CH = 128
_NC = 2
_NS = 16
_NW = _NC * _NS


def _mm(x, w):
    return jnp.dot(x, w, preferred_element_type=F32)


def _relu(x):
    return jnp.maximum(x, 0.0)


def _sig(x):
    return 0.5 * jnp.tanh(0.5 * x) + 0.5


def _tc_call(body, grid, data_ins, weight_ins, outs):
    in_specs = (
        [pl.BlockSpec((a.shape[0] // grid, a.shape[1]), lambda i: (i, 0))
         for a in data_ins]
        + [pl.BlockSpec(w.shape, lambda i: (0, 0)) for w in weight_ins]
    )
    out_specs = [pl.BlockSpec((r // grid, w), lambda i: (i, 0))
                 for (r, w) in outs]
    out_shape = [jax.ShapeDtypeStruct((r, w), F32) for (r, w) in outs]
    return pl.pallas_call(
        body, grid=(grid,), in_specs=in_specs, out_specs=out_specs,
        out_shape=out_shape,
    )(*data_ins, *weight_ins)


def _edge1_body(ax, gsp, gdp,
                ew1, eb1, ew2, eb2, ew3, eb3,
                we, b1, w2, b2, w3, b3, wh,
                e1_o, z_o):
    x = _relu(_mm(ax[...], ew1[...]) + eb1[...])
    x = _relu(_mm(x, ew2[...]) + eb2[...])
    e0 = _sig(_mm(x, ew3[...]) + eb3[...])
    h = _relu(_mm(e0, we[...]) + gsp[...] + gdp[...] + b1[...])
    h = _relu(_mm(h, w2[...]) + b2[...])
    e1 = _sig(_mm(h, w3[...]) + b3[...])
    e1_o[...] = e1
    z_o[...] = _mm(e1, wh[...])


def _edge2_body(e_in, gsp, gdp, we, b1, w2, b2, w3, b3, wh,
                e_o, z_o):
    h = _relu(_mm(e_in[...], we[...]) + gsp[...] + gdp[...] + b1[...])
    h = _relu(_mm(h, w2[...]) + b2[...])
    e2 = _sig(_mm(h, w3[...]) + b3[...])
    e_o[...] = e2
    z_o[...] = _mm(e2, wh[...])


def _dec_body(e_in, gqsp, gqdp, wde, bd1, wd2, bd2, wd3, bd3, p_o):
    d = _relu(_mm(e_in[...], wde[...]) + gqsp[...] + gqdp[...] + bd1[...])
    d = _relu(_mm(d, wd2[...]) + bd2[...])
    p_o[...] = _sig(_mm(d, wd3[...]) + bd3[...])


def _nodeA_body(cf, nw1, nb1, nw2, nb2, nw3, nb3, ws, wd,
                n0_o, ps_o, pd_o):
    x = _relu(_mm(cf[...], nw1[...]) + nb1[...])
    x = _relu(_mm(x, nw2[...]) + nb2[...])
    n0 = _sig(_mm(x, nw3[...]) + nb3[...])
    n0_o[...] = n0
    ps_o[...] = _mm(n0, ws[...])
    pd_o[...] = _mm(n0, wd[...])


def _nodeU_body(n_in, sa, sb, da, db, wn, b1, w2, b2, w3, b3, ws, wd,
                n_o, ps_o, pd_o):
    dg = jnp.maximum(da[...] + db[...], 1.0)
    hw = jnp.concatenate([sa[...] / dg, sb[...] / dg], axis=1)
    g = _relu(_mm(n_in[...], wn[...]) + hw + b1[...])
    g = _relu(_mm(g, w2[...]) + b2[...])
    nn = _sig(_mm(g, w3[...]) + b3[...])
    n_o[...] = nn
    ps_o[...] = _mm(nn, ws[...])
    pd_o[...] = _mm(nn, wd[...])


def _sc_mesh():
    return plsc.VectorSubcoreMesh(core_axis_name="c", subcore_axis_name="s",
                                  num_cores=_NC, num_subcores=_NS)


_SC_PARAMS = pltpu.CompilerParams(use_tc_tiling_on_sc=False)


@functools.lru_cache(maxsize=None)
def _make_gather(n_edges, width, k):
    nchunks = n_edges // _CH
    nblocks = nchunks // k
    max_iters = (nblocks + _NW - 1) // _NW

    @functools.partial(
        pl.kernel, mesh=_sc_mesh(), compiler_params=_SC_PARAMS,
        out_type=(jax.ShapeDtypeStruct((nchunks, _CH, width), F32),
                  jax.ShapeDtypeStruct((nchunks, _CH, width), F32)),
        scratch_types=[
            pltpu.VMEM((k, _CH), jnp.int32),
            pltpu.VMEM((k, _CH), jnp.int32),
            pltpu.VMEM((k, _CH, width), F32),
            pltpu.VMEM((k, _CH, width), F32),
            pltpu.SemaphoreType.DMA,
        ],
    )
    def gather_k(ts_hbm, td_hbm, src_hbm, dst_hbm, gs_hbm, gd_hbm,
                 i1, i2, g1, g2, sem):
        wid = lax.axis_index("s") * _NC + lax.axis_index("c")

        @pl.loop(0, max_iters)
        def _(it):
            b = wid + it * _NW

            @pl.when(b < nblocks)
            def _():
                row = b * k
                pltpu.sync_copy(src_hbm.at[pl.ds(row, k), :], i1)
                pltpu.sync_copy(dst_hbm.at[pl.ds(row, k), :], i2)
                cps = []
                for j in range(k):
                    cps.append(
                        pltpu.async_copy(ts_hbm.at[i1.at[j]], g1.at[j], sem))
                    cps.append(
                        pltpu.async_copy(td_hbm.at[i2.at[j]], g2.at[j], sem))
                for cp in cps:
                    cp.wait()
                pltpu.sync_copy(g1, gs_hbm.at[pl.ds(row, k)])
                pltpu.sync_copy(g2, gd_hbm.at[pl.ds(row, k)])

    return gather_k


@functools.lru_cache(maxsize=None)
def _make_scatter(n_nodes, n_edges, k):
    nchunks = n_edges // _CH
    nblocks = nchunks // k
    max_iters = (nblocks + _NS - 1) // _NS
    rpt = n_nodes // _NS

    @functools.partial(
        pl.kernel, mesh=_sc_mesh(), compiler_params=_SC_PARAMS,
        out_type=jax.ShapeDtypeStruct((_NC, n_nodes, 8), F32),
        scratch_types=[
            pltpu.VMEM((k, _CH), jnp.int32),
            pltpu.VMEM((k, _CH, 8), F32),
            pltpu.VMEM_SHARED((n_nodes, 8), F32),
            pltpu.SemaphoreType.DMA,
        ],
    )
    def scatter_k(z_hbm, dst_hbm, zero_hbm, out_hbm, idx, pay, acc, sem):
        cid = lax.axis_index("c")
        tid = lax.axis_index("s")
        pltpu.sync_copy(zero_hbm.at[pl.ds(tid * rpt, rpt), :],
                        acc.at[pl.ds(tid * rpt, rpt), :])
        plsc.subcore_barrier()

        @pl.loop(0, max_iters)
        def _(it):
            b = tid + it * _NS

            @pl.when(b < nblocks)
            def _():
                row = b * k
                pltpu.sync_copy(dst_hbm.at[pl.ds(row, k), :], idx)

                @pl.when(cid == 0)
                def _():
                    pltpu.sync_copy(
                        z_hbm.at[pl.ds(row, k), :, pl.ds(0, 8)], pay)

                @pl.when(cid == 1)
                def _():
                    pltpu.sync_copy(
                        z_hbm.at[pl.ds(row, k), :, pl.ds(8, 8)], pay)

                cps = [pltpu.async_copy(pay.at[j], acc.at[idx.at[j]], sem,
                                        add=True)
                       for j in range(k)]
                for cp in cps:
                    cp.wait()

        plsc.subcore_barrier()
        pltpu.sync_copy(acc.at[pl.ds(tid * rpt, rpt), :],
                        out_hbm.at[cid, pl.ds(tid * rpt, rpt), :])

    return scatter_k


@functools.lru_cache(maxsize=None)
def _make_degree(n_nodes, n_edges, k):
    nchunks = n_edges // _CH
    nblocks = nchunks // k
    hblocks = nblocks // _NC
    max_iters = (hblocks + _NS - 1) // _NS
    rpt = n_nodes // _NS

    @functools.partial(
        pl.kernel, mesh=_sc_mesh(), compiler_params=_SC_PARAMS,
        out_type=jax.ShapeDtypeStruct((_NC, n_nodes, 8), F32),
        scratch_types=[
            pltpu.VMEM((k, _CH), jnp.int32),
            pltpu.VMEM((_CH, 8), F32),
            pltpu.VMEM_SHARED((n_nodes, 8), F32),
            pltpu.SemaphoreType.DMA,
        ],
    )
    def degree_k(dst_hbm, zero_hbm, ones_hbm, out_hbm, idx, ones, acc, sem):
        cid = lax.axis_index("c")
        tid = lax.axis_index("s")
        pltpu.sync_copy(zero_hbm.at[pl.ds(tid * rpt, rpt), :],
                        acc.at[pl.ds(tid * rpt, rpt), :])
        pltpu.sync_copy(ones_hbm, ones)
        plsc.subcore_barrier()

        @pl.loop(0, max_iters)
        def _(it):
            b = cid * hblocks + tid + it * _NS

            @pl.when(b < (cid + 1) * hblocks)
            def _():
                row = b * k
                pltpu.sync_copy(dst_hbm.at[pl.ds(row, k), :], idx)
                cps = [pltpu.async_copy(ones, acc.at[idx.at[j]], sem,
                                        add=True)
                       for j in range(k)]
                for cp in cps:
                    cp.wait()

        plsc.subcore_barrier()
        pltpu.sync_copy(acc.at[pl.ds(tid * rpt, rpt), :],
                        out_hbm.at[cid, pl.ds(tid * rpt, rpt), :])

    return degree_k


def _b(bias):
    return bias.reshape(1, -1)


def kernel(C, F, A, SP1, SP0, params, edge_index):
    n_nodes = C.shape[0]
    n_edges = A.shape[0]
    nchunks = n_edges // _CH
    src = edge_index[0].astype(jnp.int32)
    dst = edge_index[1].astype(jnp.int32)
    src2 = src.reshape(nchunks, _CH)
    dst2 = dst.reshape(nchunks, _CH)
    zero8 = jnp.zeros((n_nodes, 8), F32)
    ones8 = jnp.ones((_CH, 8), F32)

    (enW1, enb1), (enW2, enb2), (enW3, enb3) = params['enc_n']
    (eeW1, eeb1), (eeW2, eeb2), (eeW3, eeb3) = params['enc_e']
    (c1eW1, c1eb1), (c1eW2, c1eb2), (c1eW3, c1eb3) = params['c1_e']
    (c1nW1, c1nb1), (c1nW2, c1nb2), (c1nW3, c1nb3) = params['c1_n']
    (c3eW1, c3eb1), (c3eW2, c3eb2), (c3eW3, c3eb3) = params['c3_e']
    (c3nW1, c3nb1), (c3nW2, c3nb2), (c3nW3, c3nb3) = params['c3_n']
    (dW1, db1), (dW2, db2), (dW3, db3) = params['dec']

    c1e_We, c1e_Ws, c1e_Wd = c1eW1[:64], c1eW1[64:128], c1eW1[128:]
    c3e_We, c3e_Ws, c3e_Wd = c3eW1[:64], c3eW1[64:128], c3eW1[128:]
    c1n_Wn, c1n_Wh = c1nW1[:64], c1nW1[64:]
    c3n_Wn, c3n_Wh = c3nW1[:64], c3nW1[64:]
    d_We, d_Ws, d_Wd = dW1[:64], dW1[64:128], dW1[128:]

    gather16 = _make_gather(n_edges, 16, 10)
    gather64 = _make_gather(n_edges, 64, 4)
    scatter = _make_scatter(n_nodes, n_edges, 10)
    degree = _make_degree(n_nodes, n_edges, 10)

    eg = n_edges // _BE
    ng = n_nodes // _BN
    er = n_edges // 8
    kron = jnp.kron
    i8 = jnp.eye(8, dtype=F32)

    def _bd(w):
        return kron(i8, w)

    def _bb(bias):
        return jnp.tile(bias.reshape(1, -1), (1, 8))

    deg = degree(dst2, zero8, ones8)
    dega, degb = deg[0], deg[1]

    n0, ps1, pd1 = _tc_call(
        _nodeA_body, ng, [jnp.concatenate([C, F], axis=1)],
        [enW1, _b(enb1), enW2, _b(enb2), enW3, _b(enb3), c1e_Ws, c1e_Wd],
        [(n_nodes, 64), (n_nodes, 16), (n_nodes, 16)])

    gs1, gd1 = gather16(ps1, pd1, src2, dst2)

    ax8 = jnp.concatenate([A, SP1, SP0], axis=1).reshape(er, 24)
    e1, z1 = _tc_call(
        _edge1_body, eg,
        [ax8, gs1.reshape(er, 128), gd1.reshape(er, 128)],
        [_bd(eeW1), _bb(eeb1), _bd(eeW2), _bb(eeb2), _bd(eeW3), _bb(eeb3),
         _bd(c1e_We), _bb(c1eb1), _bd(c1eW2), _bb(c1eb2), _bd(c1eW3),
         _bb(c1eb3), _bd(c1n_Wh)],
        [(er, 512), (er, 128)])

    s1 = scatter(z1.reshape(nchunks, _CH, 16), dst2, zero8)

    n1, ps2, pd2 = _tc_call(
        _nodeU_body, ng, [n0, s1[0], s1[1], dega, degb],
        [c1n_Wn, _b(c1nb1), c1nW2, _b(c1nb2), c1nW3, _b(c1nb3),
         c3e_Ws, c3e_Wd],
        [(n_nodes, 64), (n_nodes, 16), (n_nodes, 16)])

    gs2, gd2 = gather16(ps2, pd2, src2, dst2)

    e2, z2 = _tc_call(
        _edge2_body, eg,
        [e1, gs2.reshape(er, 128), gd2.reshape(er, 128)],
        [_bd(c3e_We), _bb(c3eb1), _bd(c3eW2), _bb(c3eb2), _bd(c3eW3),
         _bb(c3eb3), _bd(c3n_Wh)],
        [(er, 512), (er, 128)])

    s2 = scatter(z2.reshape(nchunks, _CH, 16), dst2, zero8)

    _n2, qs, qd = _tc_call(
        _nodeU_body, ng, [n1, s2[0], s2[1], dega, degb],
        [c3n_Wn, _b(c3nb1), c3nW2, _b(c3nb2), c3nW3, _b(c3nb3),
         d_Ws, d_Wd],
        [(n_nodes, 64), (n_nodes, 64), (n_nodes, 64)])

    gqs, gqd = gather64(qs, qd, src2, dst2)

    (p_out,) = _tc_call(
        _dec_body, eg,
        [e2, gqs.reshape(er, 512), gqd.reshape(er, 512)],
        [_bd(d_We), _bb(db1), _bd(dW2), _bb(db2), _bd(dW3), _bb(db3)],
        [(er, 8)])

    return p_out.reshape(n_edges)


# speedup vs baseline: 5.7644x; 1.8473x over previous
import functools

import jax
import jax.numpy as jnp
from jax import lax
from jax.experimental import pallas as pl
from jax.experimental.pallas import tpu as pltpu
from jax.experimental.pallas import tpu_sc as plsc

F32 = jnp.float32

_BE = 6400
_BN = 5000
_CH = 128
_NC = 2
_NS = 16
_NW = _NC * _NS


def _mm(x, w):
    return jnp.dot(x, w, preferred_element_type=F32)


def _relu(x):
    return jnp.maximum(x, 0.0)


def _sig(x):
    return 0.5 * jnp.tanh(0.5 * x) + 0.5


def _tc_call(body, grid, data_ins, weight_ins, outs):
    in_specs = (
        [pl.BlockSpec((a.shape[0] // grid, a.shape[1]), lambda i: (i, 0))
         for a in data_ins]
        + [pl.BlockSpec(w.shape, lambda i: (0, 0)) for w in weight_ins]
    )
    out_specs = [pl.BlockSpec((r // grid, w), lambda i: (i, 0))
                 for (r, w) in outs]
    out_shape = [jax.ShapeDtypeStruct((r, w), F32) for (r, w) in outs]
    return pl.pallas_call(
        body, grid=(grid,), in_specs=in_specs, out_specs=out_specs,
        out_shape=out_shape,
    )(*data_ins, *weight_ins)


def _edge1_body(a8, sp18, sp08, gsp, gdp,
                ewa, ewb, ewc, eb1, ew2, eb2, ew3, eb3,
                we, b1, w2, b2, w3, b3, wh,
                e1_o, z_o):
    x = _relu(_mm(a8[...], ewa[...]) + _mm(sp18[...], ewb[...])
              + _mm(sp08[...], ewc[...]) + eb1[...])
    x = _relu(_mm(x, ew2[...]) + eb2[...])
    e0 = _sig(_mm(x, ew3[...]) + eb3[...])
    h = _relu(_mm(e0, we[...]) + gsp[...] + gdp[...] + b1[...])
    h = _relu(_mm(h, w2[...]) + b2[...])
    e1 = _sig(_mm(h, w3[...]) + b3[...])
    e1_o[...] = e1
    z_o[...] = _mm(e1, wh[...])


def _edge2_body(e_in, gsp, gdp, we, b1, w2, b2, w3, b3, wh,
                e_o, z_o):
    h = _relu(_mm(e_in[...], we[...]) + gsp[...] + gdp[...] + b1[...])
    h = _relu(_mm(h, w2[...]) + b2[...])
    e2 = _sig(_mm(h, w3[...]) + b3[...])
    e_o[...] = e2
    z_o[...] = _mm(e2, wh[...])


def _dec_body(e_in, gqsp, gqdp, wde, bd1, wd2, bd2, wd3, bd3, p_o):
    d = _relu(_mm(e_in[...], wde[...]) + gqsp[...] + gqdp[...] + bd1[...])
    d = _relu(_mm(d, wd2[...]) + bd2[...])
    p_o[...] = _sig(_mm(d, wd3[...]) + bd3[...])


def _nodeA_body(c, f, nwa, nwb, nb1, nw2, nb2, nw3, nb3, ws, wd,
                n0_o, ps_o, pd_o):
    x = _relu(_mm(c[...], nwa[...]) + _mm(f[...], nwb[...]) + nb1[...])
    x = _relu(_mm(x, nw2[...]) + nb2[...])
    n0 = _sig(_mm(x, nw3[...]) + nb3[...])
    n0_o[...] = n0
    ps_o[...] = _mm(n0, ws[...])
    pd_o[...] = _mm(n0, wd[...])


def _nodeU_body(n_in, sa, sb, da, db, wn, b1, w2, b2, w3, b3, ws, wd,
                n_o, ps_o, pd_o):
    dg = jnp.maximum(da[...] + db[...], 1.0)
    hw = jnp.concatenate([sa[...] / dg, sb[...] / dg], axis=1)
    g = _relu(_mm(n_in[...], wn[...]) + hw + b1[...])
    g = _relu(_mm(g, w2[...]) + b2[...])
    nn = _sig(_mm(g, w3[...]) + b3[...])
    n_o[...] = nn
    ps_o[...] = _mm(nn, ws[...])
    pd_o[...] = _mm(nn, wd[...])


def _sc_mesh():
    return plsc.VectorSubcoreMesh(core_axis_name="c", subcore_axis_name="s",
                                  num_cores=_NC, num_subcores=_NS)


_SC_PARAMS = pltpu.CompilerParams(use_tc_tiling_on_sc=False)


@functools.lru_cache(maxsize=None)
def _make_gather(n_edges, width, k):
    nchunks = n_edges // _CH
    nblocks = nchunks // k
    max_iters = (nblocks + _NW - 1) // _NW

    @functools.partial(
        pl.kernel, mesh=_sc_mesh(), compiler_params=_SC_PARAMS,
        out_type=(jax.ShapeDtypeStruct((nchunks, _CH, width), F32),
                  jax.ShapeDtypeStruct((nchunks, _CH, width), F32)),
        scratch_types=[
            pltpu.VMEM((k, _CH), jnp.int32),
            pltpu.VMEM((k, _CH), jnp.int32),
            pltpu.VMEM((k, _CH, width), F32),
            pltpu.VMEM((k, _CH, width), F32),
            pltpu.SemaphoreType.DMA,
        ],
    )
    def gather_k(ts_hbm, td_hbm, src_hbm, dst_hbm, gs_hbm, gd_hbm,
                 i1, i2, g1, g2, sem):
        wid = lax.axis_index("s") * _NC + lax.axis_index("c")

        @pl.loop(0, max_iters)
        def _(it):
            b = wid + it * _NW

            @pl.when(b < nblocks)
            def _():
                row = b * k
                pltpu.sync_copy(src_hbm.at[pl.ds(row, k), :], i1)
                pltpu.sync_copy(dst_hbm.at[pl.ds(row, k), :], i2)
                cps = []
                for j in range(k):
                    cps.append(
                        pltpu.async_copy(ts_hbm.at[i1.at[j]], g1.at[j], sem))
                    cps.append(
                        pltpu.async_copy(td_hbm.at[i2.at[j]], g2.at[j], sem))
                for cp in cps:
                    cp.wait()
                pltpu.sync_copy(g1, gs_hbm.at[pl.ds(row, k)])
                pltpu.sync_copy(g2, gd_hbm.at[pl.ds(row, k)])

    return gather_k


@functools.lru_cache(maxsize=None)
def _make_scatter(n_nodes, n_edges, k):
    nchunks = n_edges // _CH
    nblocks = nchunks // k
    max_iters = (nblocks + _NS - 1) // _NS
    rpt = n_nodes // _NS

    @functools.partial(
        pl.kernel, mesh=_sc_mesh(), compiler_params=_SC_PARAMS,
        out_type=jax.ShapeDtypeStruct((_NC, n_nodes, 8), F32),
        scratch_types=[
            pltpu.VMEM((k, _CH), jnp.int32),
            pltpu.VMEM((k, _CH, 8), F32),
            pltpu.VMEM_SHARED((n_nodes, 8), F32),
            pltpu.SemaphoreType.DMA,
        ],
    )
    def scatter_k(z_hbm, dst_hbm, zero_hbm, out_hbm, idx, pay, acc, sem):
        cid = lax.axis_index("c")
        tid = lax.axis_index("s")
        pltpu.sync_copy(zero_hbm.at[pl.ds(tid * rpt, rpt), :],
                        acc.at[pl.ds(tid * rpt, rpt), :])
        plsc.subcore_barrier()

        @pl.loop(0, max_iters)
        def _(it):
            b = tid + it * _NS

            @pl.when(b < nblocks)
            def _():
                row = b * k
                pltpu.sync_copy(dst_hbm.at[pl.ds(row, k), :], idx)

                @pl.when(cid == 0)
                def _():
                    pltpu.sync_copy(
                        z_hbm.at[pl.ds(row, k), :, pl.ds(0, 8)], pay)

                @pl.when(cid == 1)
                def _():
                    pltpu.sync_copy(
                        z_hbm.at[pl.ds(row, k), :, pl.ds(8, 8)], pay)

                cps = [pltpu.async_copy(pay.at[j], acc.at[idx.at[j]], sem,
                                        add=True)
                       for j in range(k)]
                for cp in cps:
                    cp.wait()

        plsc.subcore_barrier()
        pltpu.sync_copy(acc.at[pl.ds(tid * rpt, rpt), :],
                        out_hbm.at[cid, pl.ds(tid * rpt, rpt), :])

    return scatter_k


@functools.lru_cache(maxsize=None)
def _make_degree(n_nodes, n_edges, k):
    nchunks = n_edges // _CH
    nblocks = nchunks // k
    hblocks = nblocks // _NC
    max_iters = (hblocks + _NS - 1) // _NS
    rpt = n_nodes // _NS

    @functools.partial(
        pl.kernel, mesh=_sc_mesh(), compiler_params=_SC_PARAMS,
        out_type=jax.ShapeDtypeStruct((_NC, n_nodes, 8), F32),
        scratch_types=[
            pltpu.VMEM((k, _CH), jnp.int32),
            pltpu.VMEM((_CH, 8), F32),
            pltpu.VMEM_SHARED((n_nodes, 8), F32),
            pltpu.SemaphoreType.DMA,
        ],
    )
    def degree_k(dst_hbm, zero_hbm, ones_hbm, out_hbm, idx, ones, acc, sem):
        cid = lax.axis_index("c")
        tid = lax.axis_index("s")
        pltpu.sync_copy(zero_hbm.at[pl.ds(tid * rpt, rpt), :],
                        acc.at[pl.ds(tid * rpt, rpt), :])
        pltpu.sync_copy(ones_hbm, ones)
        plsc.subcore_barrier()

        @pl.loop(0, max_iters)
        def _(it):
            b = cid * hblocks + tid + it * _NS

            @pl.when(b < (cid + 1) * hblocks)
            def _():
                row = b * k
                pltpu.sync_copy(dst_hbm.at[pl.ds(row, k), :], idx)
                cps = [pltpu.async_copy(ones, acc.at[idx.at[j]], sem,
                                        add=True)
                       for j in range(k)]
                for cp in cps:
                    cp.wait()

        plsc.subcore_barrier()
        pltpu.sync_copy(acc.at[pl.ds(tid * rpt, rpt), :],
                        out_hbm.at[cid, pl.ds(tid * rpt, rpt), :])

    return degree_k


def _b(bias):
    return bias.reshape(1, -1)


def kernel(C, F, A, SP1, SP0, params, edge_index):
    n_nodes = C.shape[0]
    n_edges = A.shape[0]
    nchunks = n_edges // _CH
    src = edge_index[0].astype(jnp.int32)
    dst = edge_index[1].astype(jnp.int32)
    src2 = src.reshape(nchunks, _CH)
    dst2 = dst.reshape(nchunks, _CH)
    zero8 = jnp.zeros((n_nodes, 8), F32)
    ones8 = jnp.ones((_CH, 8), F32)

    (enW1, enb1), (enW2, enb2), (enW3, enb3) = params['enc_n']
    (eeW1, eeb1), (eeW2, eeb2), (eeW3, eeb3) = params['enc_e']
    (c1eW1, c1eb1), (c1eW2, c1eb2), (c1eW3, c1eb3) = params['c1_e']
    (c1nW1, c1nb1), (c1nW2, c1nb2), (c1nW3, c1nb3) = params['c1_n']
    (c3eW1, c3eb1), (c3eW2, c3eb2), (c3eW3, c3eb3) = params['c3_e']
    (c3nW1, c3nb1), (c3nW2, c3nb2), (c3nW3, c3nb3) = params['c3_n']
    (dW1, db1), (dW2, db2), (dW3, db3) = params['dec']

    c1e_We, c1e_Ws, c1e_Wd = c1eW1[:64], c1eW1[64:128], c1eW1[128:]
    c3e_We, c3e_Ws, c3e_Wd = c3eW1[:64], c3eW1[64:128], c3eW1[128:]
    c1n_Wn, c1n_Wh = c1nW1[:64], c1nW1[64:]
    c3n_Wn, c3n_Wh = c3nW1[:64], c3nW1[64:]
    d_We, d_Ws, d_Wd = dW1[:64], dW1[64:128], dW1[128:]

    gather16 = _make_gather(n_edges, 16, 10)
    gather64 = _make_gather(n_edges, 64, 4)
    scatter = _make_scatter(n_nodes, n_edges, 10)
    degree = _make_degree(n_nodes, n_edges, 10)

    eg = n_edges // _BE
    ng = n_nodes // _BN
    er = n_edges // 8
    kron = jnp.kron
    i8 = jnp.eye(8, dtype=F32)

    def _bd(w):
        return kron(i8, w)

    def _bb(bias):
        return jnp.tile(bias.reshape(1, -1), (1, 8))

    deg = degree(dst2, zero8, ones8)
    dega, degb = deg[0], deg[1]

    n0, ps1, pd1 = _tc_call(
        _nodeA_body, ng, [C, F],
        [enW1[0:1], enW1[1:2], _b(enb1), enW2, _b(enb2), enW3, _b(enb3),
         c1e_Ws, c1e_Wd],
        [(n_nodes, 64), (n_nodes, 16), (n_nodes, 16)])

    gs1, gd1 = gather16(ps1, pd1, src2, dst2)

    e1, z1 = _tc_call(
        _edge1_body, eg,
        [A.reshape(er, 8), SP1.reshape(er, 8), SP0.reshape(er, 8),
         gs1.reshape(er, 128), gd1.reshape(er, 128)],
        [_bd(eeW1[0:1]), _bd(eeW1[1:2]), _bd(eeW1[2:3]), _bb(eeb1),
         _bd(eeW2), _bb(eeb2), _bd(eeW3), _bb(eeb3),
         _bd(c1e_We), _bb(c1eb1), _bd(c1eW2), _bb(c1eb2), _bd(c1eW3),
         _bb(c1eb3), _bd(c1n_Wh)],
        [(er, 512), (er, 128)])

    s1 = scatter(z1.reshape(nchunks, _CH, 16), dst2, zero8)

    n1, ps2, pd2 = _tc_call(
        _nodeU_body, ng, [n0, s1[0], s1[1], dega, degb],
        [c1n_Wn, _b(c1nb1), c1nW2, _b(c1nb2), c1nW3, _b(c1nb3),
         c3e_Ws, c3e_Wd],
        [(n_nodes, 64), (n_nodes, 16), (n_nodes, 16)])

    gs2, gd2 = gather16(ps2, pd2, src2, dst2)

    e2, z2 = _tc_call(
        _edge2_body, eg,
        [e1, gs2.reshape(er, 128), gd2.reshape(er, 128)],
        [_bd(c3e_We), _bb(c3eb1), _bd(c3eW2), _bb(c3eb2), _bd(c3eW3),
         _bb(c3eb3), _bd(c3n_Wh)],
        [(er, 512), (er, 128)])

    s2 = scatter(z2.reshape(nchunks, _CH, 16), dst2, zero8)

    _n2, qs, qd = _tc_call(
        _nodeU_body, ng, [n1, s2[0], s2[1], dega, degb],
        [c3n_Wn, _b(c3nb1), c3nW2, _b(c3nb2), c3nW3, _b(c3nb3),
         d_Ws, d_Wd],
        [(n_nodes, 64), (n_nodes, 64), (n_nodes, 64)])

    gqs, gqd = gather64(qs, qd, src2, dst2)

    (p_out,) = _tc_call(
        _dec_body, eg,
        [e2, gqs.reshape(er, 512), gqd.reshape(er, 512)],
        [_bd(d_We), _bb(db1), _bd(dW2), _bb(db2), _bd(dW3), _bb(db3)],
        [(er, 8)])

    return p_out.reshape(n_edges)


# speedup vs baseline: 5.8989x; 1.0233x over previous
import functools

import jax
import jax.numpy as jnp
from jax import lax
from jax.experimental import pallas as pl
from jax.experimental.pallas import tpu as pltpu
from jax.experimental.pallas import tpu_sc as plsc

F32 = jnp.float32

_BE = 6400
_BN = 5000
_CH = 128
_NC = 2
_NS = 16
_NW = _NC * _NS


def _mm(x, w):
    return jnp.dot(x, w, preferred_element_type=F32)


def _relu(x):
    return jnp.maximum(x, 0.0)


def _sig(x):
    return 0.5 * jnp.tanh(0.5 * x) + 0.5


def _tc_call(body, grid, data_ins, weight_ins, outs):
    in_specs = (
        [pl.BlockSpec((a.shape[0] // grid, a.shape[1]), lambda i: (i, 0))
         for a in data_ins]
        + [pl.BlockSpec(w.shape, lambda i: (0, 0)) for w in weight_ins]
    )
    out_specs = [pl.BlockSpec((r // grid, w), lambda i: (i, 0))
                 for (r, w) in outs]
    out_shape = [jax.ShapeDtypeStruct((r, w), F32) for (r, w) in outs]
    return pl.pallas_call(
        body, grid=(grid,), in_specs=in_specs, out_specs=out_specs,
        out_shape=out_shape,
    )(*data_ins, *weight_ins)


def _edge1_body(a8, sp18, sp08, gsp, gdp,
                ewa, ewb, ewc, eb1, ew2, eb2, ew3, eb3,
                we, b1, w2, b2, w3, b3, wh,
                e1_o, z_o):
    x = _relu(_mm(a8[...], ewa[...]) + _mm(sp18[...], ewb[...])
              + _mm(sp08[...], ewc[...]) + eb1[...])
    x = _relu(_mm(x, ew2[...]) + eb2[...])
    e0 = _sig(_mm(x, ew3[...]) + eb3[...])
    h = _relu(_mm(e0, we[...]) + gsp[...] + gdp[...] + b1[...])
    h = _relu(_mm(h, w2[...]) + b2[...])
    e1 = _sig(_mm(h, w3[...]) + b3[...])
    e1_o[...] = e1
    z_o[...] = _mm(e1, wh[...])


def _edge2_body(e_in, gsp, gdp, we, b1, w2, b2, w3, b3, wh,
                e_o, z_o):
    h = _relu(_mm(e_in[...], we[...]) + gsp[...] + gdp[...] + b1[...])
    h = _relu(_mm(h, w2[...]) + b2[...])
    e2 = _sig(_mm(h, w3[...]) + b3[...])
    e_o[...] = e2
    z_o[...] = _mm(e2, wh[...])


def _dec_body(e_in, gqsp, gqdp, wde, bd1, wd2, bd2, wd3, bd3, p_o):
    d = _relu(_mm(e_in[...], wde[...]) + gqsp[...] + gqdp[...] + bd1[...])
    d = _relu(_mm(d, wd2[...]) + bd2[...])
    p_o[...] = _sig(_mm(d, wd3[...]) + bd3[...])


def _nodeA_body(c, f, nwa, nwb, nb1, nw2, nb2, nw3, nb3, ws, wd,
                n0_o, ps_o, pd_o):
    x = _relu(_mm(c[...], nwa[...]) + _mm(f[...], nwb[...]) + nb1[...])
    x = _relu(_mm(x, nw2[...]) + nb2[...])
    n0 = _sig(_mm(x, nw3[...]) + nb3[...])
    n0_o[...] = n0
    ps_o[...] = _mm(n0, ws[...])
    pd_o[...] = _mm(n0, wd[...])


def _nodeU_body(n_in, sa, sb, da, db, wn, b1, w2, b2, w3, b3, ws, wd,
                n_o, ps_o, pd_o):
    dg = jnp.maximum(da[...] + db[...], 1.0)
    hw = jnp.concatenate([sa[...] / dg, sb[...] / dg], axis=1)
    g = _relu(_mm(n_in[...], wn[...]) + hw + b1[...])
    g = _relu(_mm(g, w2[...]) + b2[...])
    nn = _sig(_mm(g, w3[...]) + b3[...])
    n_o[...] = nn
    ps_o[...] = _mm(nn, ws[...])
    pd_o[...] = _mm(nn, wd[...])


def _sc_mesh():
    return plsc.VectorSubcoreMesh(core_axis_name="c", subcore_axis_name="s",
                                  num_cores=_NC, num_subcores=_NS)


_SC_PARAMS = pltpu.CompilerParams(use_tc_tiling_on_sc=False)


@functools.lru_cache(maxsize=None)
def _make_gather(n_edges, width, k):
    nchunks = n_edges // _CH
    nblocks = nchunks // k
    max_iters = (nblocks + _NW - 1) // _NW

    @functools.partial(
        pl.kernel, mesh=_sc_mesh(), compiler_params=_SC_PARAMS,
        out_type=(jax.ShapeDtypeStruct((nchunks, _CH, width), F32),
                  jax.ShapeDtypeStruct((nchunks, _CH, width), F32)),
        scratch_types=[
            pltpu.VMEM((k, _CH), jnp.int32),
            pltpu.VMEM((k, _CH), jnp.int32),
            pltpu.VMEM((k, _CH, width), F32),
            pltpu.VMEM((k, _CH, width), F32),
            pltpu.SemaphoreType.DMA,
        ],
    )
    def gather_k(ts_hbm, td_hbm, src_hbm, dst_hbm, gs_hbm, gd_hbm,
                 i1, i2, g1, g2, sem):
        wid = lax.axis_index("s") * _NC + lax.axis_index("c")

        @pl.loop(0, max_iters)
        def _(it):
            b = wid + it * _NW

            @pl.when(b < nblocks)
            def _():
                row = b * k
                pltpu.sync_copy(src_hbm.at[pl.ds(row, k), :], i1)
                pltpu.sync_copy(dst_hbm.at[pl.ds(row, k), :], i2)
                cps = []
                for j in range(k):
                    cps.append(
                        pltpu.async_copy(ts_hbm.at[i1.at[j]], g1.at[j], sem))
                    cps.append(
                        pltpu.async_copy(td_hbm.at[i2.at[j]], g2.at[j], sem))
                for cp in cps:
                    cp.wait()
                pltpu.sync_copy(g1, gs_hbm.at[pl.ds(row, k)])
                pltpu.sync_copy(g2, gd_hbm.at[pl.ds(row, k)])

    return gather_k


@functools.lru_cache(maxsize=None)
def _make_scatter(n_nodes, n_edges, k):
    nchunks = n_edges // _CH
    nblocks = nchunks // k
    max_iters = (nblocks + _NS - 1) // _NS
    rpt = n_nodes // _NS

    @functools.partial(
        pl.kernel, mesh=_sc_mesh(), compiler_params=_SC_PARAMS,
        out_type=jax.ShapeDtypeStruct((_NC, n_nodes, 8), F32),
        scratch_types=[
            pltpu.VMEM((k, _CH), jnp.int32),
            pltpu.VMEM((k, _CH, 8), F32),
            pltpu.VMEM_SHARED((n_nodes, 8), F32),
            pltpu.SemaphoreType.DMA,
        ],
    )
    def scatter_k(z_hbm, dst_hbm, zero_hbm, out_hbm, idx, pay, acc, sem):
        cid = lax.axis_index("c")
        tid = lax.axis_index("s")
        pltpu.sync_copy(zero_hbm.at[pl.ds(tid * rpt, rpt), :],
                        acc.at[pl.ds(tid * rpt, rpt), :])
        plsc.subcore_barrier()

        @pl.loop(0, max_iters)
        def _(it):
            b = tid + it * _NS

            @pl.when(b < nblocks)
            def _():
                row = b * k
                pltpu.sync_copy(dst_hbm.at[pl.ds(row, k), :], idx)

                @pl.when(cid == 0)
                def _():
                    pltpu.sync_copy(
                        z_hbm.at[pl.ds(row, k), :, pl.ds(0, 8)], pay)

                @pl.when(cid == 1)
                def _():
                    pltpu.sync_copy(
                        z_hbm.at[pl.ds(row, k), :, pl.ds(8, 8)], pay)

                cps = [pltpu.async_copy(pay.at[j], acc.at[idx.at[j]], sem,
                                        add=True)
                       for j in range(k)]
                for cp in cps:
                    cp.wait()

        plsc.subcore_barrier()
        pltpu.sync_copy(acc.at[pl.ds(tid * rpt, rpt), :],
                        out_hbm.at[cid, pl.ds(tid * rpt, rpt), :])

    return scatter_k


@functools.lru_cache(maxsize=None)
def _make_degree(n_nodes, n_edges, k):
    nchunks = n_edges // _CH
    nblocks = nchunks // k
    hblocks = nblocks // _NC
    max_iters = (hblocks + _NS - 1) // _NS
    rpt = n_nodes // _NS

    @functools.partial(
        pl.kernel, mesh=_sc_mesh(), compiler_params=_SC_PARAMS,
        out_type=jax.ShapeDtypeStruct((_NC, n_nodes, 8), F32),
        scratch_types=[
            pltpu.VMEM((k, _CH), jnp.int32),
            pltpu.VMEM((_CH, 8), F32),
            pltpu.VMEM_SHARED((n_nodes, 8), F32),
            pltpu.SemaphoreType.DMA,
        ],
    )
    def degree_k(dst_hbm, zero_hbm, ones_hbm, out_hbm, idx, ones, acc, sem):
        cid = lax.axis_index("c")
        tid = lax.axis_index("s")
        pltpu.sync_copy(zero_hbm.at[pl.ds(tid * rpt, rpt), :],
                        acc.at[pl.ds(tid * rpt, rpt), :])
        pltpu.sync_copy(ones_hbm, ones)
        plsc.subcore_barrier()

        @pl.loop(0, max_iters)
        def _(it):
            b = cid * hblocks + tid + it * _NS

            @pl.when(b < (cid + 1) * hblocks)
            def _():
                row = b * k
                pltpu.sync_copy(dst_hbm.at[pl.ds(row, k), :], idx)
                cps = [pltpu.async_copy(ones, acc.at[idx.at[j]], sem,
                                        add=True)
                       for j in range(k)]
                for cp in cps:
                    cp.wait()

        plsc.subcore_barrier()
        pltpu.sync_copy(acc.at[pl.ds(tid * rpt, rpt), :],
                        out_hbm.at[cid, pl.ds(tid * rpt, rpt), :])

    return degree_k


def _b(bias):
    return bias.reshape(1, -1)


def kernel(C, F, A, SP1, SP0, params, edge_index):
    n_nodes = C.shape[0]
    n_edges = A.shape[0]
    nchunks = n_edges // _CH
    src = edge_index[0].astype(jnp.int32)
    dst = edge_index[1].astype(jnp.int32)
    src2 = src.reshape(nchunks, _CH)
    dst2 = dst.reshape(nchunks, _CH)
    zero8 = jnp.zeros((n_nodes, 8), F32)
    ones8 = jnp.ones((_CH, 8), F32)

    (enW1, enb1), (enW2, enb2), (enW3, enb3) = params['enc_n']
    (eeW1, eeb1), (eeW2, eeb2), (eeW3, eeb3) = params['enc_e']
    (c1eW1, c1eb1), (c1eW2, c1eb2), (c1eW3, c1eb3) = params['c1_e']
    (c1nW1, c1nb1), (c1nW2, c1nb2), (c1nW3, c1nb3) = params['c1_n']
    (c3eW1, c3eb1), (c3eW2, c3eb2), (c3eW3, c3eb3) = params['c3_e']
    (c3nW1, c3nb1), (c3nW2, c3nb2), (c3nW3, c3nb3) = params['c3_n']
    (dW1, db1), (dW2, db2), (dW3, db3) = params['dec']

    c1e_We, c1e_Ws, c1e_Wd = c1eW1[:64], c1eW1[64:128], c1eW1[128:]
    c3e_We, c3e_Ws, c3e_Wd = c3eW1[:64], c3eW1[64:128], c3eW1[128:]
    c1n_Wn, c1n_Wh = c1nW1[:64], c1nW1[64:]
    c3n_Wn, c3n_Wh = c3nW1[:64], c3nW1[64:]
    d_We, d_Ws, d_Wd = dW1[:64], dW1[64:128], dW1[128:]

    gather16 = _make_gather(n_edges, 16, 10)
    gather64 = _make_gather(n_edges, 64, 5)
    scatter = _make_scatter(n_nodes, n_edges, 20)
    degree = _make_degree(n_nodes, n_edges, 20)

    eg = n_edges // _BE
    ng = n_nodes // _BN
    er = n_edges // 8
    kron = jnp.kron
    i8 = jnp.eye(8, dtype=F32)

    def _bd(w):
        return kron(i8, w)

    def _bb(bias):
        return jnp.tile(bias.reshape(1, -1), (1, 8))

    deg = degree(dst2, zero8, ones8)
    dega, degb = deg[0], deg[1]

    n0, ps1, pd1 = _tc_call(
        _nodeA_body, ng, [C, F],
        [enW1[0:1], enW1[1:2], _b(enb1), enW2, _b(enb2), enW3, _b(enb3),
         c1e_Ws, c1e_Wd],
        [(n_nodes, 64), (n_nodes, 16), (n_nodes, 16)])

    gs1, gd1 = gather16(ps1, pd1, src2, dst2)

    e1, z1 = _tc_call(
        _edge1_body, eg,
        [A.reshape(er, 8), SP1.reshape(er, 8), SP0.reshape(er, 8),
         gs1.reshape(er, 128), gd1.reshape(er, 128)],
        [_bd(eeW1[0:1]), _bd(eeW1[1:2]), _bd(eeW1[2:3]), _bb(eeb1),
         _bd(eeW2), _bb(eeb2), _bd(eeW3), _bb(eeb3),
         _bd(c1e_We), _bb(c1eb1), _bd(c1eW2), _bb(c1eb2), _bd(c1eW3),
         _bb(c1eb3), _bd(c1n_Wh)],
        [(er, 512), (er, 128)])

    s1 = scatter(z1.reshape(nchunks, _CH, 16), dst2, zero8)

    n1, ps2, pd2 = _tc_call(
        _nodeU_body, ng, [n0, s1[0], s1[1], dega, degb],
        [c1n_Wn, _b(c1nb1), c1nW2, _b(c1nb2), c1nW3, _b(c1nb3),
         c3e_Ws, c3e_Wd],
        [(n_nodes, 64), (n_nodes, 16), (n_nodes, 16)])

    gs2, gd2 = gather16(ps2, pd2, src2, dst2)

    e2, z2 = _tc_call(
        _edge2_body, eg,
        [e1, gs2.reshape(er, 128), gd2.reshape(er, 128)],
        [_bd(c3e_We), _bb(c3eb1), _bd(c3eW2), _bb(c3eb2), _bd(c3eW3),
         _bb(c3eb3), _bd(c3n_Wh)],
        [(er, 512), (er, 128)])

    s2 = scatter(z2.reshape(nchunks, _CH, 16), dst2, zero8)

    _n2, qs, qd = _tc_call(
        _nodeU_body, ng, [n1, s2[0], s2[1], dega, degb],
        [c3n_Wn, _b(c3nb1), c3nW2, _b(c3nb2), c3nW3, _b(c3nb3),
         d_Ws, d_Wd],
        [(n_nodes, 64), (n_nodes, 64), (n_nodes, 64)])

    gqs, gqd = gather64(qs, qd, src2, dst2)

    (p_out,) = _tc_call(
        _dec_body, eg,
        [e2, gqs.reshape(er, 512), gqd.reshape(er, 512)],
        [_bd(d_We), _bb(db1), _bd(dW2), _bb(db2), _bd(dW3), _bb(db3)],
        [(er, 8)])

    return p_out.reshape(n_edges)
